# baseline (speedup 1.0000x reference)
import functools
import jax
import jax.numpy as jnp
from jax import lax
from jax.experimental import pallas as pl
from jax.experimental.pallas import tpu as pltpu
from jax.experimental.pallas import tpu_sc as plsc

VOCAB = 1000
N_TOK = 1024 * 50
VPAD = 1024
LSE_PAD = 1024

_info = plsc.get_sparse_core_info()
NC, NS = _info.num_cores, _info.num_subcores
NW = NC * NS
TOK_W = N_TOK // NW
CHUNK = 32
NCHUNK = TOK_W // CHUNK


def _lse_body(tab_ref, out_ref):
    t = tab_ref[...]
    m = jnp.max(t, axis=1, keepdims=True)
    s = jnp.sum(jnp.exp(t - m), axis=1, keepdims=True)
    lse = m[:, 0] + jnp.log(s[:, 0])
    out_ref[...] = jnp.concatenate(
        [lse, jnp.zeros((LSE_PAD - VOCAB,), jnp.float32)])[:, None]


_lse_call = pl.pallas_call(
    _lse_body,
    out_shape=jax.ShapeDtypeStruct((LSE_PAD, 1), jnp.float32),
)


def _finalize_body(part_ref, out_ref):
    out_ref[...] = jnp.sum(part_ref[...]).reshape(1, 1) * (1.0 / N_TOK)


_finalize_call = pl.pallas_call(
    _finalize_body,
    out_shape=jax.ShapeDtypeStruct((1, 1), jnp.float32),
)


def _relayout_body(in_ref, out_ref):
    for k in range(8):
        w = min(128, VOCAB - 128 * k)
        out_ref[0, :, pl.ds(128 * k, w)] = in_ref[:, k, :w]


_relayout_call = pl.pallas_call(
    _relayout_body,
    grid=(1024,),
    in_specs=[pl.BlockSpec((50, 8, 128), lambda i: (i, 0, 0))],
    out_specs=pl.BlockSpec((1, 50, VOCAB), lambda i: (i, 0, 0)),
    out_shape=jax.ShapeDtypeStruct((1024, 50, VOCAB), jnp.float32),
)


_sc_mesh = plsc.VectorSubcoreMesh(core_axis_name="c", subcore_axis_name="s")


@functools.partial(
    pl.kernel,
    mesh=_sc_mesh,
    compiler_params=pltpu.CompilerParams(use_tc_tiling_on_sc=True),
    out_type=[
        jax.ShapeDtypeStruct((N_TOK, 8, 128), jnp.float32),
        jax.ShapeDtypeStruct((NW, 16), jnp.float32),
    ],
    scratch_types=[
        pltpu.VMEM((NCHUNK, CHUNK), jnp.int32),
        pltpu.VMEM((NCHUNK, CHUNK), jnp.int32),
        pltpu.VMEM((LSE_PAD,), jnp.float32),
        pltpu.VMEM((CHUNK, 8, 128), jnp.float32),
        pltpu.VMEM((CHUNK, 8, 128), jnp.float32),
        pltpu.VMEM((16,), jnp.float32),
        pltpu.SemaphoreType.DMA,
        pltpu.SemaphoreType.DMA,
        pltpu.SemaphoreType.DMA,
        pltpu.SemaphoreType.DMA,
    ],
)
def _sc_gather(table_hbm, x3_hbm, y3_hbm, lse_hbm,
               logits_hbm, part_hbm,
               x_v, y_v, lse_v, rows_a, rows_b, acc_v,
               gsem_a, gsem_b, wsem_a, wsem_b):
    cid = lax.axis_index("c")
    sid = lax.axis_index("s")
    wid = sid * NC + cid
    base = wid * TOK_W

    pltpu.sync_copy(x3_hbm.at[wid], x_v)
    pltpu.sync_copy(y3_hbm.at[wid], y_v)
    pltpu.sync_copy(lse_hbm, lse_v)
    acc_v[...] = jnp.zeros((16,), jnp.float32)

    bufs = (rows_a, rows_b)
    gsems = (gsem_a, gsem_b)
    wsems = (wsem_a, wsem_b)
    iota = lax.iota(jnp.int32, 16)

    def drain_wb(p):
        pltpu.make_async_copy(bufs[p], logits_hbm.at[pl.ds(base, CHUNK)],
                              wsems[p]).wait()

    def drain_gather(p):
        pltpu.make_async_copy(table_hbm.at[pl.ds(0, CHUNK)], bufs[p],
                              gsems[p]).wait()

    pltpu.async_copy(table_hbm.at[x_v.at[0]], rows_a, gsem_a)

    def chunk_body(c, _):
        par = c % 2
        for p in range(2):
            @pl.when(par == p)
            def _():
                buf = bufs[p]
                @pl.when(c >= 1)
                def _():
                    drain_wb(1 - p)

                @pl.when(c + 1 < NCHUNK)
                def _():
                    pltpu.async_copy(table_hbm.at[x_v.at[c + 1]],
                                     bufs[1 - p], gsems[1 - p])

                drain_gather(p)
                zero = jnp.zeros((16,), jnp.float32)
                for g in range(CHUNK // 16):
                    xv = x_v[c, pl.ds(g * 16, 16)]
                    yv = y_v[c, pl.ds(g * 16, 16)]
                    for i in range(16):
                        xi = xv[i]
                        yi = yv[i]
                        tok = g * 16 + i
                        lse_sl = lse_v[pl.ds((xi >> 4) << 4, 16)]
                        tgt_sl = buf[tok, yi >> 7,
                                     pl.ds(((yi >> 4) & 7) << 4, 16)]
                        acc_v[...] = (
                            acc_v[...]
                            + jnp.where(iota == (xi & 15), lse_sl, zero)
                            - jnp.where(iota == (yi & 15), tgt_sl, zero))
                pltpu.async_copy(
                    buf, logits_hbm.at[pl.ds(base + c * CHUNK, CHUNK)],
                    wsems[p])
        return 0

    lax.fori_loop(0, NCHUNK, chunk_body, 0)
    drain_wb((NCHUNK - 1) % 2)

    pltpu.sync_copy(acc_v, part_hbm.at[wid])


def kernel(x, y, table):
    B, S = x.shape
    x3 = x.astype(jnp.int32).reshape(NW, NCHUNK, CHUNK)
    y3 = y.astype(jnp.int32).reshape(NW, NCHUNK, CHUNK)
    table3 = jnp.pad(table, ((0, 0), (0, VPAD - VOCAB))).reshape(VOCAB, 8, 128)
    lse = _lse_call(table).reshape(LSE_PAD)
    out3, partials = _sc_gather(table3, x3, y3, lse)
    loss = _finalize_call(partials)[0, 0]
    logits = _relayout_call(out3)
    return (logits, loss)


# speedup vs baseline: 1.8853x; 1.8853x over previous
import functools
import jax
import jax.numpy as jnp
from jax import lax
from jax.experimental import pallas as pl
from jax.experimental.pallas import tpu as pltpu
from jax.experimental.pallas import tpu_sc as plsc

VOCAB = 1000
N_TOK = 1024 * 50
VPAD = 1024
LSE_PAD = 1024

_info = plsc.get_sparse_core_info()
NC, NS = _info.num_cores, _info.num_subcores
NW = NC * NS
TOK_W = N_TOK // NW
CHUNK = 32
NCHUNK = TOK_W // CHUNK


def _lse_body(tab_ref, out_ref):
    t = tab_ref[...]
    m = jnp.max(t, axis=1, keepdims=True)
    s = jnp.sum(jnp.exp(t - m), axis=1, keepdims=True)
    lse = m[:, 0] + jnp.log(s[:, 0])
    out_ref[...] = jnp.concatenate(
        [lse, jnp.zeros((LSE_PAD - VOCAB,), jnp.float32)])[:, None]


_lse_call = pl.pallas_call(
    _lse_body,
    out_shape=jax.ShapeDtypeStruct((LSE_PAD, 1), jnp.float32),
)


def _finalize_body(part_ref, out_ref):
    out_ref[...] = jnp.sum(part_ref[...]).reshape(1, 1) * (1.0 / N_TOK)


_finalize_call = pl.pallas_call(
    _finalize_body,
    out_shape=jax.ShapeDtypeStruct((1, 1), jnp.float32),
)


_sc_mesh = plsc.VectorSubcoreMesh(core_axis_name="c", subcore_axis_name="s")


@functools.partial(
    pl.kernel,
    mesh=_sc_mesh,
    compiler_params=pltpu.CompilerParams(use_tc_tiling_on_sc=True),
    out_type=[
        jax.ShapeDtypeStruct((N_TOK, 8, 128), jnp.float32),
        jax.ShapeDtypeStruct((NW, 16), jnp.float32),
    ],
    scratch_types=[
        pltpu.VMEM((NCHUNK, CHUNK), jnp.int32),
        pltpu.VMEM((NCHUNK, CHUNK), jnp.int32),
        pltpu.VMEM((LSE_PAD,), jnp.float32),
        pltpu.VMEM((CHUNK, 8, 128), jnp.float32),
        pltpu.VMEM((CHUNK, 8, 128), jnp.float32),
        pltpu.VMEM((16,), jnp.float32),
        pltpu.SemaphoreType.DMA,
        pltpu.SemaphoreType.DMA,
        pltpu.SemaphoreType.DMA,
        pltpu.SemaphoreType.DMA,
    ],
)
def _sc_gather(table_hbm, x3_hbm, y3_hbm, lse_hbm,
               logits_hbm, part_hbm,
               x_v, y_v, lse_v, rows_a, rows_b, acc_v,
               gsem_a, gsem_b, wsem_a, wsem_b):
    cid = lax.axis_index("c")
    sid = lax.axis_index("s")
    wid = sid * NC + cid
    base = wid * TOK_W

    pltpu.sync_copy(x3_hbm.at[wid], x_v)
    pltpu.sync_copy(y3_hbm.at[wid], y_v)
    pltpu.sync_copy(lse_hbm, lse_v)
    acc_v[...] = jnp.zeros((16,), jnp.float32)

    bufs = (rows_a, rows_b)
    gsems = (gsem_a, gsem_b)
    wsems = (wsem_a, wsem_b)
    iota = lax.iota(jnp.int32, 16)

    def drain_wb(p):
        pltpu.make_async_copy(bufs[p], logits_hbm.at[pl.ds(base, CHUNK)],
                              wsems[p]).wait()

    def drain_gather(p):
        pltpu.make_async_copy(table_hbm.at[pl.ds(0, CHUNK)], bufs[p],
                              gsems[p]).wait()

    pltpu.async_copy(table_hbm.at[x_v.at[0]], rows_a, gsem_a)

    def chunk_body(c, _):
        par = c % 2
        for p in range(2):
            @pl.when(par == p)
            def _():
                buf = bufs[p]
                @pl.when(c >= 1)
                def _():
                    drain_wb(1 - p)

                @pl.when(c + 1 < NCHUNK)
                def _():
                    pltpu.async_copy(table_hbm.at[x_v.at[c + 1]],
                                     bufs[1 - p], gsems[1 - p])

                drain_gather(p)
                zero = jnp.zeros((16,), jnp.float32)
                for g in range(CHUNK // 16):
                    xv = x_v[c, pl.ds(g * 16, 16)]
                    yv = y_v[c, pl.ds(g * 16, 16)]
                    for i in range(16):
                        xi = xv[i]
                        yi = yv[i]
                        tok = g * 16 + i
                        lse_sl = lse_v[pl.ds((xi >> 4) << 4, 16)]
                        tgt_sl = buf[tok, yi >> 7,
                                     pl.ds(((yi >> 4) & 7) << 4, 16)]
                        acc_v[...] = (
                            acc_v[...]
                            + jnp.where(iota == (xi & 15), lse_sl, zero)
                            - jnp.where(iota == (yi & 15), tgt_sl, zero))
                pltpu.async_copy(
                    buf, logits_hbm.at[pl.ds(base + c * CHUNK, CHUNK)],
                    wsems[p])
        return 0

    lax.fori_loop(0, NCHUNK, chunk_body, 0)
    drain_wb((NCHUNK - 1) % 2)

    pltpu.sync_copy(acc_v, part_hbm.at[wid])


def kernel(x, y, table):
    B, S = x.shape
    x3 = x.astype(jnp.int32).reshape(NW, NCHUNK, CHUNK)
    y3 = y.astype(jnp.int32).reshape(NW, NCHUNK, CHUNK)
    table3 = jnp.pad(table, ((0, 0), (0, VPAD - VOCAB))).reshape(VOCAB, 8, 128)
    lse = _lse_call(table).reshape(LSE_PAD)
    out3, partials = _sc_gather(table3, x3, y3, lse)
    loss = _finalize_call(partials)[0, 0]
    logits = out3.reshape(B, S, 8, 128).reshape(B, S, VPAD)[:, :, :VOCAB]
    return (logits, loss)
